# baseline (speedup 1.0000x reference)
import functools

import jax
import jax.numpy as jnp
from jax import lax
from jax.experimental import pallas as pl
from jax.experimental.pallas import tpu as pltpu
from jax.experimental.pallas import tpu_sc as plsc

_C = 4
_D = 16
_CD = _C * _D


def _transpose_block(vin, vout, npairs, rids):
    for j in range(npairs):
        for p in (0, 1):
            col = jnp.full((16,), 2 * j + p, jnp.int32)
            for k in range(4):
                vout[j, pl.ds(64 * p + 16 * k, 16)] = plsc.load_gather(
                    vin, [rids[k], col])


def _make_transpose(n_ua, n_ia):
    info = plsc.get_sparse_core_info()
    nc = info.num_cores
    mesh = plsc.VectorSubcoreMesh(core_axis_name="c", subcore_axis_name="s")

    fb_ia, tail_ia = n_ia // 128, n_ia % 128
    fb_ua, tail_ua = n_ua // 128, n_ua % 128
    main_ia, extra_ia = fb_ia // 32, fb_ia % 32
    main_ua, extra_ua = fb_ua // 32, fb_ua % 32

    @functools.partial(
        pl.kernel,
        mesh=mesh,
        compiler_params=pltpu.CompilerParams(needs_layout_passes=False),
        out_type=(
            jax.ShapeDtypeStruct((n_ua // 2, 128), jnp.float32),
            jax.ShapeDtypeStruct((n_ia // 2, 128), jnp.float32),
        ),
        scratch_types=[
            pltpu.VMEM((_CD, 128), jnp.float32),
            pltpu.VMEM((_CD, 128), jnp.float32),
            pltpu.VMEM((_CD, 128), jnp.float32),
            pltpu.VMEM((_CD, 128), jnp.float32),
            pltpu.SemaphoreType.DMA,
            pltpu.SemaphoreType.DMA,
            pltpu.SemaphoreType.DMA,
            pltpu.SemaphoreType.DMA,
        ],
    )
    def tk(ua_t, ia_t, ua_tl, ia_tl, uat, iat, vin0, vin1, vout0, vout1,
           si0, si1, so0, so1):
        wid = lax.axis_index("s") * nc + lax.axis_index("c")
        vins, vouts = (vin0, vin1), (vout0, vout1)
        sis, sos = (si0, si1), (so0, so1)
        rids = [lax.iota(jnp.int32, 16) + 16 * k for k in range(4)]

        def run(tab, out, main, extra, even_main):
            nb = main + jnp.where(wid < extra, 1, 0)

            def src(m):
                return tab.at[:, pl.ds((wid + 32 * m) * 128, 128)]

            def dst(m):
                return out.at[pl.ds((wid + 32 * m) * 64, 64), :]

            for s in (0, 1):
                @pl.when(s < nb)
                def _():
                    pltpu.async_copy(src(s), vins[s], sis[s])

            def body(i, carry):
                for s in (0, 1):
                    m = i + s

                    @pl.when(m < nb)
                    def _():
                        pltpu.make_async_copy(src(m), vins[s], sis[s]).wait()

                        @pl.when(m >= 2)
                        def _():
                            pltpu.make_async_copy(vouts[s], dst(m), sos[s]).wait()

                        _transpose_block(vins[s], vouts[s], 64, rids)
                        pltpu.async_copy(vouts[s], dst(m), sos[s])

                        @pl.when(m + 2 < nb)
                        def _():
                            pltpu.async_copy(src(m + 2), vins[s], sis[s])
                return carry

            lax.fori_loop(0, even_main // 2 + 1, lambda i, c: body(2 * i, c),
                          0, unroll=False)
            for s in (0, 1):
                @pl.when(nb >= 2 - s)
                def _():
                    pltpu.make_async_copy(vouts[s], dst(0), sos[s]).wait()

        run(ia_t, iat, main_ia, extra_ia, main_ia)
        run(ua_t, uat, main_ua, extra_ua, main_ua)

        def tail(tl, out, fb, tw, worker):
            @pl.when(wid == worker)
            def _():
                pltpu.sync_copy(tl, vins[0])
                _transpose_block(vins[0], vouts[0], tw // 2, rids)
                pltpu.sync_copy(vouts[0].at[pl.ds(0, tw // 2), :],
                                out.at[pl.ds(fb * 64, tw // 2), :])

        if tail_ia:
            tail(ia_tl, iat, fb_ia, tail_ia, 0)
        if tail_ua:
            tail(ua_tl, uat, fb_ua, tail_ua, 1)

    return tk


def _make_gather(batch):
    info = plsc.get_sparse_core_info()
    nc = info.num_cores
    nw = nc * info.num_subcores
    bpw = batch // nw
    half = bpw // 2
    mesh = plsc.VectorSubcoreMesh(core_axis_name="c", subcore_axis_name="s")

    @functools.partial(
        pl.kernel,
        mesh=mesh,
        compiler_params=pltpu.CompilerParams(use_tc_tiling_on_sc=False),
        out_type=jax.ShapeDtypeStruct((12, batch, 128), jnp.float32),
        scratch_types=[
            pltpu.VMEM((24, half), jnp.int32),
            pltpu.VMEM((half, 128), jnp.float32),
            pltpu.VMEM((half, 128), jnp.float32),
            pltpu.SemaphoreType.DMA,
            pltpu.SemaphoreType.DMA,
        ],
    )
    def gk(uat, iat, idx_hbm, out, idx_v, r0, r1, s0, s1):
        wid = lax.axis_index("s") * nc + lax.axis_index("c")
        base = wid * bpw
        pltpu.sync_copy(idx_hbm.at[wid], idx_v)
        bufs, sems = (r0, r1), (s0, s1)

        def start(c):
            tab = uat if c < 8 else iat
            return pltpu.async_copy(tab.at[idx_v.at[c]],
                                    bufs[c % 2], sems[c % 2])

        cp = start(0)
        for c in range(24):
            nxt = start(c + 1) if c < 23 else None
            cp.wait()
            g, h = c // 2, c % 2
            pltpu.sync_copy(bufs[c % 2],
                            out.at[g, pl.ds(base + h * half, half), :])
            cp = nxt

    return gk


def _tc_loss_body(g_ref, par_ref, out_ref, acc_ref):
    step = pl.program_id(0)
    nsteps = pl.num_programs(0)

    @pl.when(step == 0)
    def _init():
        acc_ref[0, 0] = jnp.float32(0.0)

    def row(g):
        x = g_ref[g]
        p = par_ref[:, g:g + 1]
        return jnp.where(p > 0.5, x[:, _CD:2 * _CD], x[:, 0:_CD])

    def uc(r, c):
        return r[:, 16 * c:16 * c + 16]

    def dot16(a, b):
        return jnp.sum(a * b, axis=1, keepdims=True)

    sig = jax.nn.sigmoid
    U = [row(g) for g in range(4)]
    P = [row(g) for g in range(4, 8)]
    N = [row(g) for g in range(8, 12)]

    def l2h(x):
        return 0.5 * jnp.sum(x * x)

    total = jnp.float32(0.0)
    for i in range(3):
        ue, pe, ne = uc(U[i], i), uc(P[i], i), uc(N[i], i)
        ps = sig(dot16(ue, pe))
        ns = sig(dot16(ue, ne))
        total += 0.01 * jnp.sum(-jax.nn.log_sigmoid(ps - ns))
        total += l2h(ue) + l2h(pe) + l2h(ne)

    ue3, pe3, ne3 = uc(U[3], 3), uc(P[3], 3), uc(N[3], 3)
    g2p = sig(dot16(ue3, pe3))
    g2n = sig(dot16(ue3, ne3))
    g1p = jnp.zeros_like(g2p)
    g1n = jnp.zeros_like(g2n)
    for k in range(3):
        uek, pek, nek = uc(U[3], k), uc(P[3], k), uc(N[3], k)
        sim = sig(dot16(ue3, uek))
        g1p += sim * sig(dot16(uek, pek))
        g1n += sim * sig(dot16(uek, nek))
    ps = (g1p / 3.0) * g2p
    ns = (g1n / 3.0) * g2n
    total += jnp.sum(-jax.nn.log_sigmoid(ps - ns))
    total += l2h(ue3) + l2h(pe3) + l2h(ne3)

    acc_ref[0, 0] += total

    @pl.when(step == nsteps - 1)
    def _fin():
        out_ref[0, 0] = acc_ref[0, 0]


def _tc_loss(gat, par, batch):
    bb = 1024
    grid = batch // bb
    out = pl.pallas_call(
        _tc_loss_body,
        grid=(grid,),
        in_specs=[
            pl.BlockSpec((12, bb, 128), lambda b: (0, b, 0)),
            pl.BlockSpec((bb, 12), lambda b: (b, 0)),
        ],
        out_specs=pl.BlockSpec(memory_space=pltpu.SMEM),
        out_shape=jax.ShapeDtypeStruct((1, 1), jnp.float32),
        scratch_shapes=[pltpu.SMEM((1, 1), jnp.float32)],
    )(gat, par)
    return out[0, 0] / (batch * 5.0)


def kernel(input_u, ua_embeddings, ia_embeddings):
    n_users, c, d = ua_embeddings.shape
    n_items = ia_embeddings.shape[0]
    batch = input_u.shape[0]
    nw = 32

    users, poss, negs = [], [], []
    for i in range(_C):
        key = jax.random.fold_in(jax.random.key(42), i)
        ku, kp, kn = jax.random.split(key, 3)
        users.append(jax.random.randint(ku, (batch,), 0, n_users))
        poss.append(jax.random.randint(kp, (batch,), 0, n_items))
        negs.append(jax.random.randint(kn, (batch,), 0, n_items))

    ent = jnp.stack(users + poss + negs).astype(jnp.int32)
    pair = ent // 2
    pair = pair.reshape(12, nw, 2, batch // nw // 2).transpose(1, 0, 2, 3)
    pair = pair.reshape(nw, 24, batch // nw // 2)
    par = (ent % 2).astype(jnp.float32).T

    t_ua = jnp.transpose(ua_embeddings, (1, 2, 0)).reshape(_CD, n_users)
    t_ia = jnp.transpose(ia_embeddings, (1, 2, 0)).reshape(_CD, n_items)

    def tail_slab(t, n):
        fb, tw = n // 128, n % 128
        return jnp.pad(t[:, fb * 128:], ((0, 0), (0, 128 - tw)))

    uat, iat = _make_transpose(n_users, n_items)(
        t_ua, t_ia, tail_slab(t_ua, n_users), tail_slab(t_ia, n_items))
    gat = _make_gather(batch)(uat, iat, pair)
    return _tc_loss(gat, par, batch)


# speedup vs baseline: 1.6700x; 1.6700x over previous
import functools

import jax
import jax.numpy as jnp
from jax import lax
from jax.experimental import pallas as pl
from jax.experimental.pallas import tpu as pltpu
from jax.experimental.pallas import tpu_sc as plsc

_C = 4
_D = 16
_CD = _C * _D


def _transpose_block(vin, vout, npairs, rids):
    @plsc.parallel_loop(0, npairs, 1, unroll=8)
    def _(j):
        for p in (0, 1):
            col = jnp.full((16,), 2 * j + p, jnp.int32)
            for k in range(4):
                vout[j, pl.ds(64 * p + 16 * k, 16)] = plsc.load_gather(
                    vin, [rids[k], col])


def _make_transpose(n_ua, n_ia):
    info = plsc.get_sparse_core_info()
    nc = info.num_cores
    mesh = plsc.VectorSubcoreMesh(core_axis_name="c", subcore_axis_name="s")

    fb_ia, tail_ia = n_ia // 128, n_ia % 128
    fb_ua, tail_ua = n_ua // 128, n_ua % 128
    main_ia, extra_ia = fb_ia // 32, fb_ia % 32
    main_ua, extra_ua = fb_ua // 32, fb_ua % 32

    @functools.partial(
        pl.kernel,
        mesh=mesh,
        compiler_params=pltpu.CompilerParams(needs_layout_passes=False),
        out_type=(
            jax.ShapeDtypeStruct((n_ua // 2, 128), jnp.float32),
            jax.ShapeDtypeStruct((n_ia // 2, 128), jnp.float32),
        ),
        scratch_types=[
            pltpu.VMEM((_CD, 128), jnp.float32),
            pltpu.VMEM((_CD, 128), jnp.float32),
            pltpu.VMEM((_CD, 128), jnp.float32),
            pltpu.VMEM((_CD, 128), jnp.float32),
            pltpu.SemaphoreType.DMA,
            pltpu.SemaphoreType.DMA,
            pltpu.SemaphoreType.DMA,
            pltpu.SemaphoreType.DMA,
        ],
    )
    def tk(ua_t, ia_t, ua_tl, ia_tl, uat, iat, vin0, vin1, vout0, vout1,
           si0, si1, so0, so1):
        wid = lax.axis_index("s") * nc + lax.axis_index("c")
        vins, vouts = (vin0, vin1), (vout0, vout1)
        sis, sos = (si0, si1), (so0, so1)
        rids = [lax.iota(jnp.int32, 16) + 16 * k for k in range(4)]

        def run(tab, out, main, extra, even_main):
            nb = main + jnp.where(wid < extra, 1, 0)

            def src(m):
                return tab.at[:, pl.ds((wid + 32 * m) * 128, 128)]

            def dst(m):
                return out.at[pl.ds((wid + 32 * m) * 64, 64), :]

            for s in (0, 1):
                @pl.when(s < nb)
                def _():
                    pltpu.async_copy(src(s), vins[s], sis[s])

            def body(i, carry):
                for s in (0, 1):
                    m = i + s

                    @pl.when(m < nb)
                    def _():
                        pltpu.make_async_copy(src(m), vins[s], sis[s]).wait()

                        @pl.when(m >= 2)
                        def _():
                            pltpu.make_async_copy(vouts[s], dst(m), sos[s]).wait()

                        _transpose_block(vins[s], vouts[s], 64, rids)
                        pltpu.async_copy(vouts[s], dst(m), sos[s])

                        @pl.when(m + 2 < nb)
                        def _():
                            pltpu.async_copy(src(m + 2), vins[s], sis[s])
                return carry

            lax.fori_loop(0, even_main // 2 + 1, lambda i, c: body(2 * i, c),
                          0, unroll=False)
            for s in (0, 1):
                @pl.when(nb >= 2 - s)
                def _():
                    pltpu.make_async_copy(vouts[s], dst(0), sos[s]).wait()

        run(ia_t, iat, main_ia, extra_ia, main_ia)
        run(ua_t, uat, main_ua, extra_ua, main_ua)

        def tail(tl, out, fb, tw, worker):
            @pl.when(wid == worker)
            def _():
                pltpu.sync_copy(tl, vins[0])
                _transpose_block(vins[0], vouts[0], tw // 2, rids)
                pltpu.sync_copy(vouts[0].at[pl.ds(0, tw // 2), :],
                                out.at[pl.ds(fb * 64, tw // 2), :])

        if tail_ia:
            tail(ia_tl, iat, fb_ia, tail_ia, 0)
        if tail_ua:
            tail(ua_tl, uat, fb_ua, tail_ua, 1)

    return tk


def _make_gather(batch):
    info = plsc.get_sparse_core_info()
    nc = info.num_cores
    nw = nc * info.num_subcores
    bpw = batch // nw
    half = bpw // 2
    mesh = plsc.VectorSubcoreMesh(core_axis_name="c", subcore_axis_name="s")

    @functools.partial(
        pl.kernel,
        mesh=mesh,
        compiler_params=pltpu.CompilerParams(use_tc_tiling_on_sc=False),
        out_type=jax.ShapeDtypeStruct((12, batch, 128), jnp.float32),
        scratch_types=[
            pltpu.VMEM((24, half), jnp.int32),
            pltpu.VMEM((half, 128), jnp.float32),
            pltpu.VMEM((half, 128), jnp.float32),
            pltpu.SemaphoreType.DMA,
            pltpu.SemaphoreType.DMA,
        ],
    )
    def gk(uat, iat, idx_hbm, out, idx_v, r0, r1, s0, s1):
        wid = lax.axis_index("s") * nc + lax.axis_index("c")
        base = wid * bpw
        pltpu.sync_copy(idx_hbm.at[wid], idx_v)
        bufs, sems = (r0, r1), (s0, s1)

        def start(c):
            tab = uat if c < 8 else iat
            return pltpu.async_copy(tab.at[idx_v.at[c]],
                                    bufs[c % 2], sems[c % 2])

        cp = start(0)
        for c in range(24):
            nxt = start(c + 1) if c < 23 else None
            cp.wait()
            g, h = c // 2, c % 2
            pltpu.sync_copy(bufs[c % 2],
                            out.at[g, pl.ds(base + h * half, half), :])
            cp = nxt

    return gk


def _tc_loss_body(g_ref, par_ref, out_ref, acc_ref):
    step = pl.program_id(0)
    nsteps = pl.num_programs(0)

    @pl.when(step == 0)
    def _init():
        acc_ref[0, 0] = jnp.float32(0.0)

    def row(g):
        x = g_ref[g]
        p = par_ref[:, g:g + 1]
        return jnp.where(p > 0.5, x[:, _CD:2 * _CD], x[:, 0:_CD])

    def uc(r, c):
        return r[:, 16 * c:16 * c + 16]

    def dot16(a, b):
        return jnp.sum(a * b, axis=1, keepdims=True)

    sig = jax.nn.sigmoid
    U = [row(g) for g in range(4)]
    P = [row(g) for g in range(4, 8)]
    N = [row(g) for g in range(8, 12)]

    def l2h(x):
        return 0.5 * jnp.sum(x * x)

    total = jnp.float32(0.0)
    for i in range(3):
        ue, pe, ne = uc(U[i], i), uc(P[i], i), uc(N[i], i)
        ps = sig(dot16(ue, pe))
        ns = sig(dot16(ue, ne))
        total += 0.01 * jnp.sum(-jax.nn.log_sigmoid(ps - ns))
        total += l2h(ue) + l2h(pe) + l2h(ne)

    ue3, pe3, ne3 = uc(U[3], 3), uc(P[3], 3), uc(N[3], 3)
    g2p = sig(dot16(ue3, pe3))
    g2n = sig(dot16(ue3, ne3))
    g1p = jnp.zeros_like(g2p)
    g1n = jnp.zeros_like(g2n)
    for k in range(3):
        uek, pek, nek = uc(U[3], k), uc(P[3], k), uc(N[3], k)
        sim = sig(dot16(ue3, uek))
        g1p += sim * sig(dot16(uek, pek))
        g1n += sim * sig(dot16(uek, nek))
    ps = (g1p / 3.0) * g2p
    ns = (g1n / 3.0) * g2n
    total += jnp.sum(-jax.nn.log_sigmoid(ps - ns))
    total += l2h(ue3) + l2h(pe3) + l2h(ne3)

    acc_ref[0, 0] += total

    @pl.when(step == nsteps - 1)
    def _fin():
        out_ref[0, 0] = acc_ref[0, 0]


def _tc_loss(gat, par, batch):
    bb = 1024
    grid = batch // bb
    out = pl.pallas_call(
        _tc_loss_body,
        grid=(grid,),
        in_specs=[
            pl.BlockSpec((12, bb, 128), lambda b: (0, b, 0)),
            pl.BlockSpec((bb, 12), lambda b: (b, 0)),
        ],
        out_specs=pl.BlockSpec(memory_space=pltpu.SMEM),
        out_shape=jax.ShapeDtypeStruct((1, 1), jnp.float32),
        scratch_shapes=[pltpu.SMEM((1, 1), jnp.float32)],
    )(gat, par)
    return out[0, 0] / (batch * 5.0)


def kernel(input_u, ua_embeddings, ia_embeddings):
    n_users, c, d = ua_embeddings.shape
    n_items = ia_embeddings.shape[0]
    batch = input_u.shape[0]
    nw = 32

    users, poss, negs = [], [], []
    for i in range(_C):
        key = jax.random.fold_in(jax.random.key(42), i)
        ku, kp, kn = jax.random.split(key, 3)
        users.append(jax.random.randint(ku, (batch,), 0, n_users))
        poss.append(jax.random.randint(kp, (batch,), 0, n_items))
        negs.append(jax.random.randint(kn, (batch,), 0, n_items))

    ent = jnp.stack(users + poss + negs).astype(jnp.int32)
    pair = ent // 2
    pair = pair.reshape(12, nw, 2, batch // nw // 2).transpose(1, 0, 2, 3)
    pair = pair.reshape(nw, 24, batch // nw // 2)
    par = (ent % 2).astype(jnp.float32).T

    t_ua = jnp.transpose(ua_embeddings, (1, 2, 0)).reshape(_CD, n_users)
    t_ia = jnp.transpose(ia_embeddings, (1, 2, 0)).reshape(_CD, n_items)

    def tail_slab(t, n):
        fb, tw = n // 128, n % 128
        return jnp.pad(t[:, fb * 128:], ((0, 0), (0, 128 - tw)))

    uat, iat = _make_transpose(n_users, n_items)(
        t_ua, t_ia, tail_slab(t_ua, n_users), tail_slab(t_ia, n_items))
    gat = _make_gather(batch)(uat, iat, pair)
    return _tc_loss(gat, par, batch)
